# baseline (speedup 1.0000x reference)
import jax
import jax.numpy as jnp
from jax.experimental import pallas as pl
from jax.experimental.pallas import tpu as pltpu


_BT = 512


def _rvq_block_kernel(x_ref, cb2_ref, cbt_ref, quant_ref, codes_ref, loss_ref,
                      cnormc_ref, chk_ref):
    first = (pl.program_id(0) == 0) & (pl.program_id(1) == 0)
    n_q = cb2_ref.shape[0]
    K = cb2_ref.shape[1]
    D = cb2_ref.shape[2]

    @pl.when(first)
    def _init():
        loss_ref[...] = jnp.zeros_like(loss_ref)
        cn_rows = []
        for i in range(n_q):
            e_t = cbt_ref[i]
            cn_rows.append(jnp.sum(e_t * e_t, axis=0, keepdims=True))
            hi = e_t.astype(jnp.bfloat16)
            r1 = e_t - hi.astype(jnp.float32)
            mid = r1.astype(jnp.bfloat16)
            r2 = r1 - mid.astype(jnp.float32)
            chk_ref[i, 0:D, :] = hi
            chk_ref[i, D:2 * D, :] = mid
            chk_ref[i, 2 * D:3 * D, :] = r2.astype(jnp.bfloat16)
        cnormc_ref[...] = jnp.concatenate(cn_rows, axis=0).T

    flat = x_ref[0]
    residual = flat
    sq = flat * flat
    rows = flat.shape[1]
    iota = jax.lax.broadcasted_iota(jnp.int32, (K, rows), 0)
    dn = (((1,), (0,)), ((), ()))
    for i in range(n_q):
        a = jnp.sum(sq, axis=0, keepdims=True)
        b2 = jax.lax.dot_general(cb2_ref[i], residual, dn,
                                 preferred_element_type=jnp.float32)
        w = (b2 - a) - cnormc_ref[:, i:i + 1]
        idx = jnp.argmax(w, axis=0)
        onehot = (iota == idx[None, :]).astype(jnp.float32
                                               ).astype(jnp.bfloat16)
        q3 = jax.lax.dot_general(chk_ref[i], onehot, dn,
                                 preferred_element_type=jnp.float32)
        q = (q3[0:D, :] + q3[D:2 * D, :]) + q3[2 * D:3 * D, :]
        q_st = residual + (q - residual)
        residual = residual - q_st
        sq = residual * residual
        loss_ref[:, i:i + 1] += jnp.sum(sq, axis=1, keepdims=True)
        codes_ref[0, i, :] = idx
    quant_ref[0] = flat - residual


def _rvq_call(x, cb2, cbt, interpret=False):
    B, D, T = x.shape
    n_q_s, K, _ = cb2.shape
    grid = (B, T // _BT)
    return pl.pallas_call(
        _rvq_block_kernel,
        grid=grid,
        in_specs=[
            pl.BlockSpec((1, D, _BT), lambda b, t: (b, 0, t)),
            pl.BlockSpec((n_q_s, K, D), lambda b, t: (0, 0, 0)),
            pl.BlockSpec((n_q_s, D, K), lambda b, t: (0, 0, 0)),
        ],
        out_specs=[
            pl.BlockSpec((1, D, _BT), lambda b, t: (b, 0, t)),
            pl.BlockSpec((1, n_q_s, _BT), lambda b, t: (b, 0, t)),
            pl.BlockSpec((D, n_q_s), lambda b, t: (0, 0)),
        ],
        out_shape=[
            jax.ShapeDtypeStruct((B, D, T), jnp.float32),
            jax.ShapeDtypeStruct((B, n_q_s, T), jnp.int32),
            jax.ShapeDtypeStruct((D, n_q_s), jnp.float32),
        ],
        scratch_shapes=[pltpu.VMEM((K, n_q_s), jnp.float32),
                        pltpu.VMEM((n_q_s, 3 * D, K), jnp.bfloat16)],
        interpret=interpret,
    )(x, cb2, cbt)


def kernel(x, n_q, codebooks, interpret=False):
    B, D, T = x.shape
    cb2 = codebooks * jnp.float32(2.0)
    cbt = jnp.transpose(codebooks, (0, 2, 1))
    quant, codes_bnt, loss_acc = _rvq_call(x, cb2, cbt, interpret=interpret)
    codes = jnp.transpose(codes_bnt, (1, 0, 2))
    losses = jnp.sum(loss_acc, axis=0) / (B * T * D)
    penalty = jnp.mean(losses) + (jnp.asarray(n_q) * 0).astype(losses.dtype)
    return quant, codes, penalty


# speedup vs baseline: 1.2118x; 1.2118x over previous
import jax
import jax.numpy as jnp
from jax.experimental import pallas as pl
from jax.experimental.pallas import tpu as pltpu


_BT = 2048


def _rvq_block_kernel(x_ref, cb2_ref, cbt_ref, quant_ref, codes_ref, loss_ref,
                      cnormc_ref, chk_ref):
    first = (pl.program_id(0) == 0) & (pl.program_id(1) == 0)
    n_q = cb2_ref.shape[0]
    K = cb2_ref.shape[1]
    D = cb2_ref.shape[2]

    @pl.when(first)
    def _init():
        loss_ref[...] = jnp.zeros_like(loss_ref)
        cn_rows = []
        for i in range(n_q):
            e_t = cbt_ref[i]
            cn_rows.append(jnp.sum(e_t * e_t, axis=0, keepdims=True))
            hi = e_t.astype(jnp.bfloat16)
            r1 = e_t - hi.astype(jnp.float32)
            mid = r1.astype(jnp.bfloat16)
            r2 = r1 - mid.astype(jnp.float32)
            chk_ref[i, 0:D, :] = hi
            chk_ref[i, D:2 * D, :] = mid
            chk_ref[i, 2 * D:3 * D, :] = r2.astype(jnp.bfloat16)
        cnormc_ref[...] = jnp.concatenate(cn_rows, axis=0).T

    flat = x_ref[0]
    residual = flat
    sq = flat * flat
    rows = flat.shape[1]
    iota = jax.lax.broadcasted_iota(jnp.int32, (K, rows), 0)
    dn = (((1,), (0,)), ((), ()))
    for i in range(n_q):
        a = jnp.sum(sq, axis=0, keepdims=True)
        b2 = jax.lax.dot_general(cb2_ref[i], residual, dn,
                                 preferred_element_type=jnp.float32)
        w = (b2 - a) - cnormc_ref[:, i:i + 1]
        idx = jnp.argmax(w, axis=0)
        onehot = (iota == idx[None, :]).astype(jnp.float32
                                               ).astype(jnp.bfloat16)
        q3 = jax.lax.dot_general(chk_ref[i], onehot, dn,
                                 preferred_element_type=jnp.float32)
        q = (q3[0:D, :] + q3[D:2 * D, :]) + q3[2 * D:3 * D, :]
        q_st = residual + (q - residual)
        residual = residual - q_st
        sq = residual * residual
        loss_ref[:, i:i + 1] += jnp.sum(sq, axis=1, keepdims=True)
        codes_ref[0, i, :] = idx
    quant_ref[0] = flat - residual


def _rvq_call(x, cb2, cbt, interpret=False):
    B, D, T = x.shape
    n_q_s, K, _ = cb2.shape
    grid = (B, T // _BT)
    return pl.pallas_call(
        _rvq_block_kernel,
        grid=grid,
        in_specs=[
            pl.BlockSpec((1, D, _BT), lambda b, t: (b, 0, t)),
            pl.BlockSpec((n_q_s, K, D), lambda b, t: (0, 0, 0)),
            pl.BlockSpec((n_q_s, D, K), lambda b, t: (0, 0, 0)),
        ],
        out_specs=[
            pl.BlockSpec((1, D, _BT), lambda b, t: (b, 0, t)),
            pl.BlockSpec((1, n_q_s, _BT), lambda b, t: (b, 0, t)),
            pl.BlockSpec((D, n_q_s), lambda b, t: (0, 0)),
        ],
        out_shape=[
            jax.ShapeDtypeStruct((B, D, T), jnp.float32),
            jax.ShapeDtypeStruct((B, n_q_s, T), jnp.int32),
            jax.ShapeDtypeStruct((D, n_q_s), jnp.float32),
        ],
        scratch_shapes=[pltpu.VMEM((K, n_q_s), jnp.float32),
                        pltpu.VMEM((n_q_s, 3 * D, K), jnp.bfloat16)],
        interpret=interpret,
    )(x, cb2, cbt)


def kernel(x, n_q, codebooks, interpret=False):
    B, D, T = x.shape
    cb2 = codebooks * jnp.float32(2.0)
    cbt = jnp.transpose(codebooks, (0, 2, 1))
    quant, codes_bnt, loss_acc = _rvq_call(x, cb2, cbt, interpret=interpret)
    codes = jnp.transpose(codes_bnt, (1, 0, 2))
    losses = jnp.sum(loss_acc, axis=0) / (B * T * D)
    penalty = jnp.mean(losses) + (jnp.asarray(n_q) * 0).astype(losses.dtype)
    return quant, codes, penalty


# speedup vs baseline: 1.4172x; 1.1695x over previous
import jax
import jax.numpy as jnp
from jax.experimental import pallas as pl
from jax.experimental.pallas import tpu as pltpu


_BT = 2048


def _rvq_block_kernel(x_ref, cb_ref, quant_ref, codes_ref, loss_ref,
                      cnormc_ref, chk_ref, cb2_ref):
    first = (pl.program_id(0) == 0) & (pl.program_id(1) == 0)
    n_q = cb_ref.shape[0]
    K = cb_ref.shape[1]
    D = cb_ref.shape[2]

    @pl.when(first)
    def _init():
        loss_ref[...] = jnp.zeros_like(loss_ref)
        cn_rows = []
        for i in range(n_q):
            cbi = cb_ref[i]
            cb2_ref[i] = cbi + cbi
            e_t = cbi.T
            cn_rows.append(jnp.sum(e_t * e_t, axis=0, keepdims=True))
            hi = e_t.astype(jnp.bfloat16)
            r1 = e_t - hi.astype(jnp.float32)
            mid = r1.astype(jnp.bfloat16)
            r2 = r1 - mid.astype(jnp.float32)
            chk_ref[i, 0:D, :] = hi
            chk_ref[i, D:2 * D, :] = mid
            chk_ref[i, 2 * D:3 * D, :] = r2.astype(jnp.bfloat16)
        cnormc_ref[...] = jnp.concatenate(cn_rows, axis=0).T

    flat = x_ref[0]
    residual = flat
    sq = flat * flat
    rows = flat.shape[1]
    iota = jax.lax.broadcasted_iota(jnp.int32, (K, rows), 0)
    dn = (((1,), (0,)), ((), ()))
    for i in range(n_q):
        a = jnp.sum(sq, axis=0, keepdims=True)
        b2 = jax.lax.dot_general(cb2_ref[i], residual, dn,
                                 preferred_element_type=jnp.float32)
        w = (b2 - a) - cnormc_ref[:, i:i + 1]
        idx = jnp.argmax(w, axis=0)
        onehot = (iota == idx[None, :]).astype(jnp.float32
                                               ).astype(jnp.bfloat16)
        q3 = jax.lax.dot_general(chk_ref[i], onehot, dn,
                                 preferred_element_type=jnp.float32)
        q = (q3[0:D, :] + q3[D:2 * D, :]) + q3[2 * D:3 * D, :]
        q_st = residual + (q - residual)
        residual = residual - q_st
        sq = residual * residual
        loss_ref[:, i:i + 1] += jnp.sum(sq, axis=1, keepdims=True)
        codes_ref[0, i, :] = idx
    quant_ref[0] = flat - residual


def _rvq_call(x, codebooks, interpret=False):
    B, D, T = x.shape
    n_q_s, K, _ = codebooks.shape
    grid = (B, T // _BT)
    return pl.pallas_call(
        _rvq_block_kernel,
        grid=grid,
        in_specs=[
            pl.BlockSpec((1, D, _BT), lambda b, t: (b, 0, t)),
            pl.BlockSpec((n_q_s, K, D), lambda b, t: (0, 0, 0)),
        ],
        out_specs=[
            pl.BlockSpec((1, D, _BT), lambda b, t: (b, 0, t)),
            pl.BlockSpec((1, n_q_s, _BT), lambda b, t: (b, 0, t)),
            pl.BlockSpec((D, n_q_s), lambda b, t: (0, 0)),
        ],
        out_shape=[
            jax.ShapeDtypeStruct((B, D, T), jnp.float32),
            jax.ShapeDtypeStruct((B, n_q_s, T), jnp.int32),
            jax.ShapeDtypeStruct((D, n_q_s), jnp.float32),
        ],
        scratch_shapes=[pltpu.VMEM((K, n_q_s), jnp.float32),
                        pltpu.VMEM((n_q_s, 3 * D, K), jnp.bfloat16),
                        pltpu.VMEM((n_q_s, K, D), jnp.float32)],
        interpret=interpret,
    )(x, codebooks)


def kernel(x, n_q, codebooks, interpret=False):
    B, D, T = x.shape
    quant, codes_bnt, loss_acc = _rvq_call(x, codebooks, interpret=interpret)
    codes = jnp.transpose(codes_bnt, (1, 0, 2))
    losses = jnp.sum(loss_acc, axis=0) / (B * T * D)
    penalty = jnp.mean(losses) + (jnp.asarray(n_q) * 0).astype(losses.dtype)
    return quant, codes, penalty


# speedup vs baseline: 1.4186x; 1.0010x over previous
import jax
import jax.numpy as jnp
from jax.experimental import pallas as pl
from jax.experimental.pallas import tpu as pltpu


_BT = 2048


def _rvq_block_kernel(x_ref, cb_ref, quant_ref, codes_ref, loss_ref,
                      cnormc_ref, chk_ref, cb2_ref):
    first = (pl.program_id(0) == 0) & (pl.program_id(1) == 0)
    n_q = cb_ref.shape[0]
    K = cb_ref.shape[1]
    D = cb_ref.shape[2]

    @pl.when(first)
    def _init():
        loss_ref[...] = jnp.zeros_like(loss_ref)
        cn_rows = []
        for i in range(n_q):
            cbi = cb_ref[i]
            cb2_ref[i] = cbi + cbi
            e_t = cbi.T
            cn_rows.append(jnp.sum(e_t * e_t, axis=0, keepdims=True))
            hi = e_t.astype(jnp.bfloat16)
            r1 = e_t - hi.astype(jnp.float32)
            mid = r1.astype(jnp.bfloat16)
            r2 = r1 - mid.astype(jnp.float32)
            chk_ref[i, 0:D, :] = hi
            chk_ref[i, D:2 * D, :] = mid
            chk_ref[i, 2 * D:3 * D, :] = r2.astype(jnp.bfloat16)
        cnormc_ref[...] = jnp.concatenate(cn_rows, axis=0).T

    flat = x_ref[0]
    residual = flat
    sq = flat * flat
    rows = flat.shape[1]
    iota = jax.lax.broadcasted_iota(jnp.int32, (K, rows), 0)
    dn = (((1,), (0,)), ((), ()))
    for i in range(n_q):
        a = jnp.sum(sq, axis=0, keepdims=True)
        b2 = jax.lax.dot_general(cb2_ref[i], residual, dn,
                                 preferred_element_type=jnp.float32)
        w = (b2 - a) - cnormc_ref[:, i:i + 1]
        idx = jnp.argmax(w, axis=0)
        onehot = (iota == idx[None, :]).astype(jnp.float32
                                               ).astype(jnp.bfloat16)
        q3 = jax.lax.dot_general(chk_ref[i], onehot, dn,
                                 preferred_element_type=jnp.float32)
        q = (q3[0:D, :] + q3[D:2 * D, :]) + q3[2 * D:3 * D, :]
        q_st = residual + (q - residual)
        residual = residual - q_st
        sq = residual * residual
        loss_ref[:, i:i + 1] += jnp.sum(sq, axis=1, keepdims=True)
        codes_ref[0, i, :] = idx
    quant_ref[0] = flat - residual


def _rvq_call(x, codebooks):
    B, D, T = x.shape
    n_q_s, K, _ = codebooks.shape
    grid = (B, T // _BT)
    return pl.pallas_call(
        _rvq_block_kernel,
        grid=grid,
        in_specs=[
            pl.BlockSpec((1, D, _BT), lambda b, t: (b, 0, t)),
            pl.BlockSpec((n_q_s, K, D), lambda b, t: (0, 0, 0)),
        ],
        out_specs=[
            pl.BlockSpec((1, D, _BT), lambda b, t: (b, 0, t)),
            pl.BlockSpec((1, n_q_s, _BT), lambda b, t: (b, 0, t)),
            pl.BlockSpec((D, n_q_s), lambda b, t: (0, 0)),
        ],
        out_shape=[
            jax.ShapeDtypeStruct((B, D, T), jnp.float32),
            jax.ShapeDtypeStruct((B, n_q_s, T), jnp.int32),
            jax.ShapeDtypeStruct((D, n_q_s), jnp.float32),
        ],
        scratch_shapes=[pltpu.VMEM((K, n_q_s), jnp.float32),
                        pltpu.VMEM((n_q_s, 3 * D, K), jnp.bfloat16),
                        pltpu.VMEM((n_q_s, K, D), jnp.float32)],
    )(x, codebooks)


def kernel(x, n_q, codebooks):
    B, D, T = x.shape
    quant, codes_bnt, loss_acc = _rvq_call(x, codebooks)
    codes = jnp.transpose(codes_bnt, (1, 0, 2))
    losses = jnp.sum(loss_acc, axis=0) / (B * T * D)
    penalty = jnp.mean(losses) + (jnp.asarray(n_q) * 0).astype(losses.dtype)
    return quant, codes, penalty
